# Initial kernel scaffold; baseline (speedup 1.0000x reference)
import jax
import jax.numpy as jnp
from jax.experimental import pallas as pl


def kernel(indices, table):
    raise NotImplementedError("write your pallas kernel here")


# speedup vs baseline: 2.5875x; 2.5875x over previous
import functools

import jax
import jax.numpy as jnp
from jax import lax
from jax.experimental import pallas as pl
from jax.experimental.pallas import tpu as pltpu
from jax.experimental.pallas import tpu_sc as plsc

B, S, D = 16384, 200, 64
NC, NS = 2, 16
NW = NC * NS
BLK = 128
CB = 4
ROWS = B * S
NBLK = ROWS // BLK
NBLK_W = NBLK // NW


def _body(idx_hbm, table_hbm, out_hbm, idx_v, rows_v, sem):
    wid = lax.axis_index("s") * NC + lax.axis_index("c")
    base = wid * NBLK_W

    def chunk(t, carry):
        cb = base + t * CB
        pltpu.sync_copy(idx_hbm.at[pl.ds(cb, CB)], idx_v)
        cps = [
            pltpu.async_copy(table_hbm.at[idx_v.at[j]], rows_v.at[j], sem)
            for j in range(CB)
        ]
        for c in cps:
            c.wait()
        pltpu.sync_copy(rows_v, out_hbm.at[pl.ds(cb, CB)])
        return carry

    lax.fori_loop(0, NBLK_W // CB, chunk, 0)


_mesh = plsc.VectorSubcoreMesh(core_axis_name="c", subcore_axis_name="s")

_gather = functools.partial(
    pl.kernel,
    out_type=jax.ShapeDtypeStruct((NBLK, BLK, D), jnp.float32),
    mesh=_mesh,
    scratch_types=[
        pltpu.VMEM((CB, BLK), jnp.int32),
        pltpu.VMEM((CB, BLK, D), jnp.float32),
        pltpu.SemaphoreType.DMA,
    ],
    compiler_params=pltpu.CompilerParams(use_tc_tiling_on_sc=False),
)(_body)


def kernel(indices, table):
    idx = indices.reshape(NBLK, BLK).astype(jnp.int32)
    out = _gather(idx, table.astype(jnp.float32))
    return out.reshape(B, S, D)


# speedup vs baseline: 2.5886x; 1.0004x over previous
import functools

import jax
import jax.numpy as jnp
from jax import lax
from jax.experimental import pallas as pl
from jax.experimental.pallas import tpu as pltpu
from jax.experimental.pallas import tpu_sc as plsc

B, S, D = 16384, 200, 64
NC, NS = 2, 16
NW = NC * NS
BLK = 128
NB = 5
ROWS = B * S
NBLK = ROWS // BLK
NBLK_W = NBLK // NW
U = NBLK_W // NB
CB_BYTES = NB * BLK * D * 4
IDX_BYTES = NB * BLK * 4


def _body(idx_hbm, table_hbm, out_hbm,
          idxA, idxB, rowsA, rowsB, gsem, osemA, osemB, isemA, isemB):
    wid = lax.axis_index("s") * NC + lax.axis_index("c")
    base = wid * NBLK_W

    pltpu.async_copy(idx_hbm.at[pl.ds(base, NB)], idxA, isemA)
    pltpu.async_copy(idx_hbm.at[pl.ds(base + NB, NB)], idxB, isemB)
    pltpu.async_copy(out_hbm.at[pl.ds(base, NB)], rowsA, osemA)
    pltpu.async_copy(out_hbm.at[pl.ds(base, NB)], rowsB, osemB)

    def sub(u, idx_v, rows_v, osem, isem):
        bstart = base + u * NB
        pltpu.make_async_copy(idx_hbm.at[pl.ds(base, NB)], idx_v, isem).wait()
        pltpu.make_async_copy(out_hbm.at[pl.ds(base, NB)], rows_v, osem).wait()
        hs = [
            pltpu.async_copy(table_hbm.at[idx_v.at[j]], rows_v.at[j], gsem)
            for j in range(NB)
        ]
        for h in hs:
            h.wait()
        u_pref = jnp.minimum(u + 2, U - 1)
        pltpu.async_copy(idx_hbm.at[pl.ds(base + u_pref * NB, NB)], idx_v, isem)
        pltpu.async_copy(rows_v, out_hbm.at[pl.ds(bstart, NB)], osem)

    def body(t, carry):
        sub(2 * t, idxA, rowsA, osemA, isemA)
        sub(2 * t + 1, idxB, rowsB, osemB, isemB)
        return carry

    lax.fori_loop(0, U // 2, body, 0)

    pltpu.make_async_copy(out_hbm.at[pl.ds(base, NB)], rowsA, osemA).wait()
    pltpu.make_async_copy(out_hbm.at[pl.ds(base, NB)], rowsB, osemB).wait()
    pltpu.make_async_copy(idx_hbm.at[pl.ds(base, NB)], idxA, isemA).wait()
    pltpu.make_async_copy(idx_hbm.at[pl.ds(base, NB)], idxB, isemB).wait()


_mesh = plsc.VectorSubcoreMesh(core_axis_name="c", subcore_axis_name="s")

_gather = functools.partial(
    pl.kernel,
    out_type=jax.ShapeDtypeStruct((NBLK, BLK, D), jnp.float32),
    mesh=_mesh,
    scratch_types=[
        pltpu.VMEM((NB, BLK), jnp.int32),
        pltpu.VMEM((NB, BLK), jnp.int32),
        pltpu.VMEM((NB, BLK, D), jnp.float32),
        pltpu.VMEM((NB, BLK, D), jnp.float32),
        pltpu.SemaphoreType.DMA,
        pltpu.SemaphoreType.DMA,
        pltpu.SemaphoreType.DMA,
        pltpu.SemaphoreType.DMA,
        pltpu.SemaphoreType.DMA,
    ],
    compiler_params=pltpu.CompilerParams(use_tc_tiling_on_sc=False),
)(_body)


def kernel(indices, table):
    idx = indices.reshape(NBLK, BLK).astype(jnp.int32)
    out = _gather(idx, table.astype(jnp.float32))
    return out.reshape(B, S, D)


# speedup vs baseline: 5.7925x; 2.2377x over previous
import functools

import jax
import jax.numpy as jnp
from jax import lax
from jax.experimental import pallas as pl
from jax.experimental.pallas import tpu as pltpu
from jax.experimental.pallas import tpu_sc as plsc

B, S, D = 16384, 200, 64
NC, NS = 2, 16
NW = NC * NS
BLK = 128
NB = 5
ROWS = B * S
NBLK = ROWS // BLK
NBLK_W = NBLK // NW
U = NBLK_W // NB
CB_BYTES = NB * BLK * D * 4
IDX_BYTES = NB * BLK * 4


def _body(idx_hbm, table_hbm, out_hbm,
          idxA, idxB, rowsA, rowsB, table_v, gsem, osemA, osemB, isemA, isemB):
    wid = lax.axis_index("s") * NC + lax.axis_index("c")
    base = wid * NBLK_W

    @pl.when(lax.axis_index("s") == 0)
    def _stage():
        pltpu.sync_copy(table_hbm, table_v)
    plsc.subcore_barrier()

    pltpu.async_copy(idx_hbm.at[pl.ds(base, NB)], idxA, isemA)
    pltpu.async_copy(idx_hbm.at[pl.ds(base + NB, NB)], idxB, isemB)
    pltpu.async_copy(out_hbm.at[pl.ds(base, NB)], rowsA, osemA)
    pltpu.async_copy(out_hbm.at[pl.ds(base, NB)], rowsB, osemB)

    def sub(u, idx_v, rows_v, osem, isem):
        bstart = base + u * NB
        pltpu.make_async_copy(idx_hbm.at[pl.ds(base, NB)], idx_v, isem).wait()
        pltpu.make_async_copy(out_hbm.at[pl.ds(base, NB)], rows_v, osem).wait()
        hs = [
            pltpu.async_copy(table_v.at[idx_v.at[j]], rows_v.at[j], gsem)
            for j in range(NB)
        ]
        for h in hs:
            h.wait()
        u_pref = jnp.minimum(u + 2, U - 1)
        pltpu.async_copy(idx_hbm.at[pl.ds(base + u_pref * NB, NB)], idx_v, isem)
        pltpu.async_copy(rows_v, out_hbm.at[pl.ds(bstart, NB)], osem)

    def body(t, carry):
        sub(2 * t, idxA, rowsA, osemA, isemA)
        sub(2 * t + 1, idxB, rowsB, osemB, isemB)
        return carry

    lax.fori_loop(0, U // 2, body, 0)

    pltpu.make_async_copy(out_hbm.at[pl.ds(base, NB)], rowsA, osemA).wait()
    pltpu.make_async_copy(out_hbm.at[pl.ds(base, NB)], rowsB, osemB).wait()
    pltpu.make_async_copy(idx_hbm.at[pl.ds(base, NB)], idxA, isemA).wait()
    pltpu.make_async_copy(idx_hbm.at[pl.ds(base, NB)], idxB, isemB).wait()


_mesh = plsc.VectorSubcoreMesh(core_axis_name="c", subcore_axis_name="s")

_gather = functools.partial(
    pl.kernel,
    out_type=jax.ShapeDtypeStruct((NBLK, BLK, D), jnp.float32),
    mesh=_mesh,
    scratch_types=[
        pltpu.VMEM((NB, BLK), jnp.int32),
        pltpu.VMEM((NB, BLK), jnp.int32),
        pltpu.VMEM((NB, BLK, D), jnp.float32),
        pltpu.VMEM((NB, BLK, D), jnp.float32),
        pltpu.VMEM_SHARED((65, D), jnp.float32),
        pltpu.SemaphoreType.DMA,
        pltpu.SemaphoreType.DMA,
        pltpu.SemaphoreType.DMA,
        pltpu.SemaphoreType.DMA,
        pltpu.SemaphoreType.DMA,
    ],
    compiler_params=pltpu.CompilerParams(use_tc_tiling_on_sc=False),
)(_body)


def kernel(indices, table):
    idx = indices.reshape(NBLK, BLK).astype(jnp.int32)
    out = _gather(idx, table.astype(jnp.float32))
    return out.reshape(B, S, D)
